# baseline (speedup 1.0000x reference)
import functools

import jax
import jax.numpy as jnp
from jax import lax
from jax.experimental import pallas as pl
from jax.experimental.pallas import tpu as pltpu
from jax.experimental.pallas import tpu_sc as plsc

NC = 2
NS = 16
NW = NC * NS

CW = 128


def _make_gather(B, D, chunks_per_w):
    b_per_w = chunks_per_w * CW
    mesh = plsc.VectorSubcoreMesh(core_axis_name="c", subcore_axis_name="s")

    @functools.partial(
        pl.kernel,
        out_type=jax.ShapeDtypeStruct((B, D), jnp.float32),
        mesh=mesh,
        scratch_types=[
            pltpu.VMEM((chunks_per_w, CW), jnp.int32),
            pltpu.VMEM((CW, D), jnp.float32),
            pltpu.VMEM((CW, D), jnp.float32),
            pltpu.SemaphoreType.DMA,
            pltpu.SemaphoreType.DMA,
        ],
        compiler_params=pltpu.CompilerParams(use_tc_tiling_on_sc=False),
    )
    def k(table_hbm, idx_hbm, out_hbm, idx_v, rows0, rows1, sem0, sem1):
        wid = lax.axis_index("s") * NC + lax.axis_index("c")
        base = wid * b_per_w
        pltpu.sync_copy(idx_hbm.at[wid], idx_v)

        rows = (rows0, rows1)
        sems = (sem0, sem1)

        pltpu.async_copy(table_hbm.at[idx_v.at[0]], rows0, sem0)

        def step(j, carry):
            cur = jnp.mod(j, 2)

            def do(p):
                rb, sb = rows[p], sems[p]
                ro, so = rows[1 - p], sems[1 - p]
                @pl.when(j + 1 < chunks_per_w)
                def _start():
                    pltpu.async_copy(table_hbm.at[idx_v.at[j + 1]], ro, so)
                pltpu.make_async_copy(table_hbm.at[idx_v.at[j]], rb, sb).wait()
                pltpu.sync_copy(rb, out_hbm.at[pl.ds(base + j * CW, CW)])

            @pl.when(cur == 0)
            def _even():
                do(0)

            @pl.when(cur == 1)
            def _odd():
                do(1)

            return carry

        lax.fori_loop(0, chunks_per_w, step, 0, unroll=False)

    return k


@jax.jit
def kernel(sparse_table, indices):
    n0, n1 = indices.shape
    D = sparse_table.shape[1]
    B = n0 * n1
    chunks_per_w = B // (NW * CW)
    idx = indices.reshape(NW, chunks_per_w, CW).astype(jnp.int32)
    out = _make_gather(B, D, chunks_per_w)(sparse_table, idx)
    return out.reshape(n0, n1, D)


# speedup vs baseline: 1.0135x; 1.0135x over previous
import functools

import jax
import jax.numpy as jnp
from jax import lax
from jax.experimental import pallas as pl
from jax.experimental.pallas import tpu as pltpu
from jax.experimental.pallas import tpu_sc as plsc

NC = 2
NS = 16
NW = NC * NS

CW = 128
DEPTH = 8
AHEAD = DEPTH - 2


def _make_gather(B, D, chunks_per_w):
    b_per_w = chunks_per_w * CW
    mesh = plsc.VectorSubcoreMesh(core_axis_name="c", subcore_axis_name="s")

    @functools.partial(
        pl.kernel,
        out_type=jax.ShapeDtypeStruct((B, D), jnp.float32),
        mesh=mesh,
        scratch_types=(
            [pltpu.VMEM((chunks_per_w, CW), jnp.int32)]
            + [pltpu.VMEM((CW, D), jnp.float32) for _ in range(DEPTH)]
            + [pltpu.SemaphoreType.DMA for _ in range(2 * DEPTH)]
        ),
        compiler_params=pltpu.CompilerParams(use_tc_tiling_on_sc=False),
    )
    def k(table_hbm, idx_hbm, out_hbm, idx_v, *bufs):
        rows = bufs[:DEPTH]
        gsem = bufs[DEPTH:2 * DEPTH]
        wsem = bufs[2 * DEPTH:]
        wid = lax.axis_index("s") * NC + lax.axis_index("c")
        base = wid * b_per_w
        pltpu.sync_copy(idx_hbm.at[wid], idx_v)

        def start_gather(q, s):
            pltpu.async_copy(table_hbm.at[idx_v.at[q]], rows[s], gsem[s])

        def wait_gather(q, s):
            pltpu.make_async_copy(table_hbm.at[idx_v.at[q]], rows[s], gsem[s]).wait()

        def start_write(q, s):
            pltpu.async_copy(rows[s], out_hbm.at[pl.ds(base + q * CW, CW)], wsem[s])

        def wait_write(q, s):
            pltpu.make_async_copy(
                rows[s], out_hbm.at[pl.ds(base + q * CW, CW)], wsem[s]).wait()

        for b in range(AHEAD):
            start_gather(b, b)

        @pl.loop(0, chunks_per_w, step=DEPTH)
        def _group(g):
            for b in range(DEPTH):
                q = g + b
                s = b
                wait_gather(q, s)
                start_write(q, s)
                sf = (b + AHEAD) % DEPTH

                @pl.when(q + AHEAD < chunks_per_w)
                def _refill():
                    @pl.when(q + AHEAD >= DEPTH)
                    def _drain():
                        wait_write(q + AHEAD - DEPTH, sf)
                    start_gather(q + AHEAD, sf)

        for i in range(DEPTH):
            q = chunks_per_w - DEPTH + i
            wait_write(q, q % DEPTH)

    return k


@jax.jit
def kernel(sparse_table, indices):
    n0, n1 = indices.shape
    D = sparse_table.shape[1]
    B = n0 * n1
    chunks_per_w = B // (NW * CW)
    idx = indices.reshape(NW, chunks_per_w, CW).astype(jnp.int32)
    out = _make_gather(B, D, chunks_per_w)(sparse_table, idx)
    return out.reshape(n0, n1, D)


# speedup vs baseline: 1.0151x; 1.0015x over previous
import functools

import jax
import jax.numpy as jnp
from jax import lax
from jax.experimental import pallas as pl
from jax.experimental.pallas import tpu as pltpu
from jax.experimental.pallas import tpu_sc as plsc

NC = 2
NS = 16
NW = NC * NS

CW = 128
DEPTH = 13
AHEAD = DEPTH - 2


def _make_gather(B, D, chunks_per_w):
    b_per_w = chunks_per_w * CW
    mesh = plsc.VectorSubcoreMesh(core_axis_name="c", subcore_axis_name="s")

    @functools.partial(
        pl.kernel,
        out_type=jax.ShapeDtypeStruct((B, D), jnp.float32),
        mesh=mesh,
        scratch_types=(
            [pltpu.VMEM((chunks_per_w, CW), jnp.int32)]
            + [pltpu.VMEM((CW, D), jnp.float32) for _ in range(DEPTH)]
            + [pltpu.SemaphoreType.DMA for _ in range(2 * DEPTH)]
        ),
        compiler_params=pltpu.CompilerParams(use_tc_tiling_on_sc=False),
    )
    def k(table_hbm, idx_hbm, out_hbm, idx_v, *bufs):
        rows = bufs[:DEPTH]
        gsem = bufs[DEPTH:2 * DEPTH]
        wsem = bufs[2 * DEPTH:]
        wid = lax.axis_index("s") * NC + lax.axis_index("c")
        base = wid * b_per_w
        pltpu.sync_copy(idx_hbm.at[wid], idx_v)

        def start_gather(q, s):
            pltpu.async_copy(table_hbm.at[idx_v.at[q]], rows[s], gsem[s])

        def wait_gather(q, s):
            pltpu.make_async_copy(table_hbm.at[idx_v.at[q]], rows[s], gsem[s]).wait()

        def start_write(q, s):
            pltpu.async_copy(rows[s], out_hbm.at[pl.ds(base + q * CW, CW)], wsem[s])

        def wait_write(q, s):
            pltpu.make_async_copy(
                rows[s], out_hbm.at[pl.ds(base + q * CW, CW)], wsem[s]).wait()

        for b in range(AHEAD):
            start_gather(b, b)

        @pl.loop(0, chunks_per_w, step=DEPTH)
        def _group(g):
            for b in range(DEPTH):
                q = g + b
                s = b
                wait_gather(q, s)
                start_write(q, s)
                sf = (b + AHEAD) % DEPTH

                @pl.when(q + AHEAD < chunks_per_w)
                def _refill():
                    @pl.when(q + AHEAD >= DEPTH)
                    def _drain():
                        wait_write(q + AHEAD - DEPTH, sf)
                    start_gather(q + AHEAD, sf)

        for i in range(DEPTH):
            q = chunks_per_w - DEPTH + i
            wait_write(q, q % DEPTH)

    return k


@jax.jit
def kernel(sparse_table, indices):
    n0, n1 = indices.shape
    D = sparse_table.shape[1]
    B = n0 * n1
    chunks_per_w = B // (NW * CW)
    idx = indices.reshape(NW, chunks_per_w, CW).astype(jnp.int32)
    out = _make_gather(B, D, chunks_per_w)(sparse_table, idx)
    return out.reshape(n0, n1, D)
